# baseline (speedup 1.0000x reference)
import functools

import jax
import jax.numpy as jnp
from jax import lax
from jax.experimental import pallas as pl
from jax.experimental.pallas import tpu as pltpu
from jax.experimental.pallas import tpu_sc as plsc

T = 1728
B = 32
C = 768
NMASK = 432
KEPT = T - NMASK
NROWS = KEPT * B
NW = 32
RPW = NROWS // NW
CH = 48
NCHUNK = RPW // CH
NB = 3
L = 16

_mesh = plsc.VectorSubcoreMesh(core_axis_name="c", subcore_axis_name="s")


@functools.partial(
    pl.kernel,
    mesh=_mesh,
    out_type=jax.ShapeDtypeStruct((NROWS, C), jnp.float32),
    scratch_types=[
        pltpu.VMEM((RPW,), jnp.int32),
        pltpu.VMEM((NB, CH, C), jnp.float32),
        pltpu.SemaphoreType.DMA,
        pltpu.SemaphoreType.DMA,
        pltpu.SemaphoreType.DMA,
        pltpu.SemaphoreType.DMA,
        pltpu.SemaphoreType.DMA,
        pltpu.SemaphoreType.DMA,
    ],
)
def _gather_rows(patches_hbm, fwd_hbm, out_hbm, idx_v, rows_v,
                 gsem0, gsem1, gsem2, wsem0, wsem1, wsem2):
    wid = lax.axis_index("s") * 2 + lax.axis_index("c")
    base = wid * RPW
    gsems = (gsem0, gsem1, gsem2)
    wsems = (wsem0, wsem1, wsem2)

    pltpu.sync_copy(fwd_hbm.at[pl.ds(base, RPW)], idx_v)
    for k in range(RPW // L):
        lane_j = base + k * L + lax.iota(jnp.int32, L)
        idx_v[pl.ds(k * L, L)] = idx_v[pl.ds(k * L, L)] * B + lax.rem(lane_j, B)

    gathers = [None] * NCHUNK
    writes = [None] * NCHUNK
    for c in range(NCHUNK):
        b = c % NB
        if c >= NB:
            writes[c - NB].wait()
        gathers[c] = pltpu.async_copy(
            patches_hbm.at[idx_v.at[pl.ds(c * CH, CH)]], rows_v.at[b], gsems[b])
        if c >= 1:
            p = c - 1
            gathers[p].wait()
            writes[p] = pltpu.async_copy(
                rows_v.at[p % NB], out_hbm.at[pl.ds(base + p * CH, CH)],
                wsems[p % NB])
    last = NCHUNK - 1
    gathers[last].wait()
    writes[last] = pltpu.async_copy(
        rows_v.at[last % NB], out_hbm.at[pl.ds(base + last * CH, CH)],
        wsems[last % NB])
    for p in range(max(0, NCHUNK - NB), NCHUNK):
        writes[p].wait()


def kernel(patches, forward_indexes, backward_indexes):
    patches_2d = patches.reshape(T * B, C)
    fwd_flat = forward_indexes[:KEPT].astype(jnp.int32).reshape(-1)
    out_2d = _gather_rows(patches_2d, fwd_flat)
    kept = out_2d.reshape(KEPT, B, C)
    return (kept, forward_indexes, backward_indexes)


# speedup vs baseline: 1.0096x; 1.0096x over previous
import functools

import jax
import jax.numpy as jnp
from jax import lax
from jax.experimental import pallas as pl
from jax.experimental.pallas import tpu as pltpu
from jax.experimental.pallas import tpu_sc as plsc

T = 1728
B = 32
C = 768
NMASK = 432
KEPT = T - NMASK
NROWS = KEPT * B
NW = 32
RPW = NROWS // NW
CH = 72
NCHUNK = RPW // CH
NB = 2
L = 16

_mesh = plsc.VectorSubcoreMesh(core_axis_name="c", subcore_axis_name="s")


@functools.partial(
    pl.kernel,
    mesh=_mesh,
    out_type=jax.ShapeDtypeStruct((NROWS, C), jnp.float32),
    scratch_types=[
        pltpu.VMEM((RPW,), jnp.int32),
        pltpu.VMEM((NB, CH, C), jnp.float32),
        pltpu.SemaphoreType.DMA,
        pltpu.SemaphoreType.DMA,
        pltpu.SemaphoreType.DMA,
        pltpu.SemaphoreType.DMA,
        pltpu.SemaphoreType.DMA,
        pltpu.SemaphoreType.DMA,
    ],
)
def _gather_rows(patches_hbm, fwd_hbm, out_hbm, idx_v, rows_v,
                 gsem0, gsem1, gsem2, wsem0, wsem1, wsem2):
    wid = lax.axis_index("s") * 2 + lax.axis_index("c")
    base = wid * RPW
    gsems = (gsem0, gsem1, gsem2)
    wsems = (wsem0, wsem1, wsem2)

    pltpu.sync_copy(fwd_hbm.at[pl.ds(base, RPW)], idx_v)
    for k in range(RPW // L):
        lane_j = base + k * L + lax.iota(jnp.int32, L)
        idx_v[pl.ds(k * L, L)] = idx_v[pl.ds(k * L, L)] * B + lax.rem(lane_j, B)

    gathers = [None] * NCHUNK
    writes = [None] * NCHUNK
    for c in range(NCHUNK):
        b = c % NB
        if c >= NB:
            writes[c - NB].wait()
        gathers[c] = pltpu.async_copy(
            patches_hbm.at[idx_v.at[pl.ds(c * CH, CH)]], rows_v.at[b], gsems[b])
        if c >= 1:
            p = c - 1
            gathers[p].wait()
            writes[p] = pltpu.async_copy(
                rows_v.at[p % NB], out_hbm.at[pl.ds(base + p * CH, CH)],
                wsems[p % NB])
    last = NCHUNK - 1
    gathers[last].wait()
    writes[last] = pltpu.async_copy(
        rows_v.at[last % NB], out_hbm.at[pl.ds(base + last * CH, CH)],
        wsems[last % NB])
    for p in range(max(0, NCHUNK - NB), NCHUNK):
        writes[p].wait()


def kernel(patches, forward_indexes, backward_indexes):
    patches_2d = patches.reshape(T * B, C)
    fwd_flat = forward_indexes.astype(jnp.int32).reshape(T * B)
    out_2d = _gather_rows(patches_2d, fwd_flat)
    kept = out_2d.reshape(KEPT, B, C)
    return (kept, forward_indexes, backward_indexes)


# speedup vs baseline: 1.0111x; 1.0015x over previous
import functools

import jax
import jax.numpy as jnp
from jax import lax
from jax.experimental import pallas as pl
from jax.experimental.pallas import tpu as pltpu
from jax.experimental.pallas import tpu_sc as plsc

T = 1728
B = 32
C = 768
NMASK = 432
KEPT = T - NMASK
NROWS = KEPT * B
NW = 32
RPW = NROWS // NW
CH = 72
NCHUNK = RPW // CH
NB = 2
L = 16
NGRP = RPW // L

_mesh = plsc.VectorSubcoreMesh(core_axis_name="c", subcore_axis_name="s")


@functools.partial(
    pl.kernel,
    mesh=_mesh,
    out_type=jax.ShapeDtypeStruct((NROWS, C), jnp.float32),
    scratch_types=[
        pltpu.VMEM((RPW,), jnp.int32),
        pltpu.VMEM((NB, CH, C), jnp.float32),
        pltpu.SemaphoreType.DMA,
        pltpu.SemaphoreType.DMA,
        pltpu.SemaphoreType.DMA,
        pltpu.SemaphoreType.DMA,
    ],
)
def _gather_rows(patches_hbm, fwd_hbm, out_hbm, idx_v, rows_v,
                 gsem0, gsem1, wsem0, wsem1):
    wid = lax.axis_index("s") * 2 + lax.axis_index("c")
    base = wid * RPW
    gsems = (gsem0, gsem1)
    wsems = (wsem0, wsem1)

    pltpu.sync_copy(fwd_hbm.at[pl.ds(base, RPW)], idx_v)

    def compute_idx(g_lo, g_hi):
        for k in range(g_lo, g_hi):
            lane_j = base + k * L + lax.iota(jnp.int32, L)
            idx_v[pl.ds(k * L, L)] = (
                idx_v[pl.ds(k * L, L)] * B + lax.rem(lane_j, B))

    head_groups = -(-(NB * CH) // L)
    compute_idx(0, head_groups)

    gathers = [None] * NCHUNK
    writes = [None] * NCHUNK
    for c in range(NB):
        gathers[c] = pltpu.async_copy(
            patches_hbm.at[idx_v.at[pl.ds(c * CH, CH)]], rows_v.at[c % NB],
            gsems[c % NB])

    compute_idx(head_groups, NGRP)

    for c in range(NCHUNK):
        b = c % NB
        if c >= NB:
            writes[c - NB].wait()
            gathers[c] = pltpu.async_copy(
                patches_hbm.at[idx_v.at[pl.ds(c * CH, CH)]], rows_v.at[b],
                gsems[b])
        if c >= 1:
            p = c - 1
            gathers[p].wait()
            writes[p] = pltpu.async_copy(
                rows_v.at[p % NB], out_hbm.at[pl.ds(base + p * CH, CH)],
                wsems[p % NB])
    last = NCHUNK - 1
    gathers[last].wait()
    writes[last] = pltpu.async_copy(
        rows_v.at[last % NB], out_hbm.at[pl.ds(base + last * CH, CH)],
        wsems[last % NB])
    for p in range(NCHUNK - NB, NCHUNK):
        writes[p].wait()


def kernel(patches, forward_indexes, backward_indexes):
    patches_2d = patches.reshape(T * B, C)
    fwd_flat = forward_indexes.astype(jnp.int32).reshape(T * B)
    out_2d = _gather_rows(patches_2d, fwd_flat)
    kept = out_2d.reshape(KEPT, B, C)
    return (kept, forward_indexes, backward_indexes)
